# Initial kernel scaffold; baseline (speedup 1.0000x reference)
import jax
import jax.numpy as jnp
from jax.experimental import pallas as pl


def kernel(u, v, wu0, wv0, b0, wu1, wv1, b1):
    raise NotImplementedError("write your pallas kernel here")


# speedup vs baseline: 56.7033x; 56.7033x over previous
import functools

import jax
import jax.numpy as jnp
from jax.experimental import pallas as pl
from jax.experimental.pallas import tpu as pltpu

_EPS = 1e-7


def _stats_body(x_ref, s_ref, q_ref):
    i = pl.program_id(1)

    @pl.when(i == 0)
    def _():
        s_ref[...] = jnp.zeros_like(s_ref)
        q_ref[...] = jnp.zeros_like(q_ref)

    xb = x_ref[...]
    s_ref[...] += jnp.sum(xb, axis=0, keepdims=True)
    q_ref[...] += jnp.sum(xb * xb, axis=0, keepdims=True)


def _col_stats(x, rb, cb):
    n, d = x.shape
    grid = (d // cb, n // rb)
    return pl.pallas_call(
        _stats_body,
        grid=grid,
        in_specs=[pl.BlockSpec((rb, cb), lambda c, i: (i, c))],
        out_specs=[
            pl.BlockSpec((1, cb), lambda c, i: (0, c)),
            pl.BlockSpec((1, cb), lambda c, i: (0, c)),
        ],
        out_shape=[
            jax.ShapeDtypeStruct((1, d), jnp.float32),
            jax.ShapeDtypeStruct((1, d), jnp.float32),
        ],
        compiler_params=pltpu.CompilerParams(
            dimension_semantics=("parallel", "arbitrary")),
        name="col_stats",
    )(x)


def _fused_body(u_ref, v_ref, su_ref, qu_ref, sv_ref, qv_ref,
                wv0u_ref, wv0v_ref, wu0u_ref, wu0v_ref, b0u_ref, b0v_ref,
                wv1u_ref, wv1v_ref, wu1u_ref, wu1v_ref, b1u_ref, b1v_ref,
                ou_ref, ov_ref, *, n_rows):
    inv_n = 1.0 / n_rows
    inv_nm1 = 1.0 / (n_rows - 1.0)

    su = su_ref[...]
    mu = su * inv_n
    varu = (qu_ref[...] - su * mu) * inv_nm1
    scu = 1.0 / (jnp.sqrt(jnp.maximum(varu, 0.0)) + _EPS)

    sv = sv_ref[...]
    mv = sv * inv_n
    varv = (qv_ref[...] - sv * mv) * inv_nm1
    scv = 1.0 / (jnp.sqrt(jnp.maximum(varv, 0.0)) + _EPS)

    xu = (u_ref[...] - mu) * scu
    xv = (v_ref[...] - mv) * scv

    def layer(xu, xv, wvu, wvv, wuu, wuv, bu, bv):
        h = (jnp.dot(xu, wvu, preferred_element_type=jnp.float32)
             + jnp.dot(xv, wvv, preferred_element_type=jnp.float32))
        pu = jnp.dot(h, wuu, preferred_element_type=jnp.float32) + bu
        pv = jnp.dot(h, wuv, preferred_element_type=jnp.float32) + bv
        return jnp.maximum(pu, 0.0) + xu, jnp.maximum(pv, 0.0) + xv

    xu, xv = layer(xu, xv, wv0u_ref[...], wv0v_ref[...],
                   wu0u_ref[...], wu0v_ref[...], b0u_ref[...], b0v_ref[...])
    xu, xv = layer(xu, xv, wv1u_ref[...], wv1v_ref[...],
                   wu1u_ref[...], wu1v_ref[...], b1u_ref[...], b1v_ref[...])
    ou_ref[...] = xu
    ov_ref[...] = xv


def kernel(u, v, wu0, wv0, b0, wu1, wv1, b1):
    u = u.astype(jnp.float32)
    v = v.astype(jnp.float32)
    n, xd = u.shape
    dd = v.shape[1]

    rb_stats = min(1024, n)
    su, qu = _col_stats(u, rb_stats, min(512, xd))
    sv, qv = _col_stats(v, rb_stats, min(512, dd))

    wv0u, wv0v = wv0[:, :xd].T, wv0[:, xd:].T
    wu0u, wu0v = wu0[:xd].T, wu0[xd:].T
    wv1u, wv1v = wv1[:, :xd].T, wv1[:, xd:].T
    wu1u, wu1v = wu1[:xd].T, wu1[xd:].T
    b0u, b0v = b0[:xd].reshape(1, xd), b0[xd:].reshape(1, dd)
    b1u, b1v = b1[:xd].reshape(1, xd), b1[xd:].reshape(1, dd)

    rb = min(256, n)
    grid = (n // rb,)
    full = lambda a: pl.BlockSpec(a.shape, lambda i: (0,) * a.ndim)
    ou, ov = pl.pallas_call(
        functools.partial(_fused_body, n_rows=float(n)),
        grid=grid,
        in_specs=[
            pl.BlockSpec((rb, xd), lambda i: (i, 0)),
            pl.BlockSpec((rb, dd), lambda i: (i, 0)),
            full(su), full(qu), full(sv), full(qv),
            full(wv0u), full(wv0v), full(wu0u), full(wu0v),
            full(b0u), full(b0v),
            full(wv1u), full(wv1v), full(wu1u), full(wu1v),
            full(b1u), full(b1v),
        ],
        out_specs=[
            pl.BlockSpec((rb, xd), lambda i: (i, 0)),
            pl.BlockSpec((rb, dd), lambda i: (i, 0)),
        ],
        out_shape=[
            jax.ShapeDtypeStruct((n, xd), jnp.float32),
            jax.ShapeDtypeStruct((n, dd), jnp.float32),
        ],
        compiler_params=pltpu.CompilerParams(
            dimension_semantics=("parallel",),
            vmem_limit_bytes=56 * 1024 * 1024),
        name="norm_lr_mlp",
    )(u, v, su, qu, sv, qv,
      wv0u, wv0v, wu0u, wu0v, b0u, b0v,
      wv1u, wv1v, wu1u, wu1v, b1u, b1v)
    return ou, ov


# speedup vs baseline: 71.9284x; 1.2685x over previous
import functools

import jax
import jax.numpy as jnp
from jax.experimental import pallas as pl
from jax.experimental.pallas import tpu as pltpu

_EPS = 1e-7

_CONTRACT_DIM1 = (((1,), (1,)), ((), ()))


def _stats_body(x_ref, s_ref, q_ref):
    i = pl.program_id(1)

    @pl.when(i == 0)
    def _():
        s_ref[...] = jnp.zeros_like(s_ref)
        q_ref[...] = jnp.zeros_like(q_ref)

    xb = x_ref[...]
    s_ref[...] += jnp.sum(xb, axis=0, keepdims=True)
    q_ref[...] += jnp.sum(xb * xb, axis=0, keepdims=True)


def _col_stats(x, rb, cb):
    n, d = x.shape
    grid = (d // cb, n // rb)
    return pl.pallas_call(
        _stats_body,
        grid=grid,
        in_specs=[pl.BlockSpec((rb, cb), lambda c, i: (i, c))],
        out_specs=[
            pl.BlockSpec((1, cb), lambda c, i: (0, c)),
            pl.BlockSpec((1, cb), lambda c, i: (0, c)),
        ],
        out_shape=[
            jax.ShapeDtypeStruct((1, d), jnp.float32),
            jax.ShapeDtypeStruct((1, d), jnp.float32),
        ],
        compiler_params=pltpu.CompilerParams(
            dimension_semantics=("parallel", "arbitrary")),
        name="col_stats",
    )(x)


def _dot_t(x, w):
    return jax.lax.dot_general(x, w, _CONTRACT_DIM1,
                               preferred_element_type=jnp.float32)


def _fused_body(u_ref, v_ref, su_ref, qu_ref, sv_ref, qv_ref,
                wu0_ref, wv0_ref, b0_ref, wu1_ref, wv1_ref, b1_ref,
                ou_ref, ov_ref, *, n_rows, xd):
    inv_n = 1.0 / n_rows
    inv_nm1 = 1.0 / (n_rows - 1.0)

    su = su_ref[...]
    mu = su * inv_n
    varu = (qu_ref[...] - su * mu) * inv_nm1
    scu = 1.0 / (jnp.sqrt(jnp.maximum(varu, 0.0)) + _EPS)

    sv = sv_ref[...]
    mv = sv * inv_n
    varv = (qv_ref[...] - sv * mv) * inv_nm1
    scv = 1.0 / (jnp.sqrt(jnp.maximum(varv, 0.0)) + _EPS)

    xu = (u_ref[...] - mu) * scu
    xv = (v_ref[...] - mv) * scv

    def layer(xu, xv, wu_ref, wv_ref, b_ref):
        h = _dot_t(xu, wv_ref[:, :xd]) + _dot_t(xv, wv_ref[:, xd:])
        pu = _dot_t(h, wu_ref[:xd, :]) + b_ref[:, :xd]
        pv = _dot_t(h, wu_ref[xd:, :]) + b_ref[:, xd:]
        return jnp.maximum(pu, 0.0) + xu, jnp.maximum(pv, 0.0) + xv

    xu, xv = layer(xu, xv, wu0_ref, wv0_ref, b0_ref)
    xu, xv = layer(xu, xv, wu1_ref, wv1_ref, b1_ref)
    ou_ref[...] = xu
    ov_ref[...] = xv


def kernel(u, v, wu0, wv0, b0, wu1, wv1, b1):
    u = u.astype(jnp.float32)
    v = v.astype(jnp.float32)
    n, xd = u.shape
    dd = v.shape[1]
    d = xd + dd

    rb_stats = min(2048, n)
    su, qu = _col_stats(u, rb_stats, min(512, xd))
    sv, qv = _col_stats(v, rb_stats, min(512, dd))

    b0r = b0.reshape(1, d)
    b1r = b1.reshape(1, d)

    rb = min(256, n)
    grid = (n // rb,)
    full = lambda a: pl.BlockSpec(a.shape, lambda i: (0,) * a.ndim)
    ou, ov = pl.pallas_call(
        functools.partial(_fused_body, n_rows=float(n), xd=xd),
        grid=grid,
        in_specs=[
            pl.BlockSpec((rb, xd), lambda i: (i, 0)),
            pl.BlockSpec((rb, dd), lambda i: (i, 0)),
            full(su), full(qu), full(sv), full(qv),
            full(wu0), full(wv0), full(b0r),
            full(wu1), full(wv1), full(b1r),
        ],
        out_specs=[
            pl.BlockSpec((rb, xd), lambda i: (i, 0)),
            pl.BlockSpec((rb, dd), lambda i: (i, 0)),
        ],
        out_shape=[
            jax.ShapeDtypeStruct((n, xd), jnp.float32),
            jax.ShapeDtypeStruct((n, dd), jnp.float32),
        ],
        compiler_params=pltpu.CompilerParams(
            dimension_semantics=("parallel",),
            vmem_limit_bytes=56 * 1024 * 1024),
        name="norm_lr_mlp",
    )(u, v, su, qu, sv, qv, wu0, wv0, b0r, wu1, wv1, b1r)
    return ou, ov


# speedup vs baseline: 74.4093x; 1.0345x over previous
import functools

import jax
import jax.numpy as jnp
from jax.experimental import pallas as pl
from jax.experimental.pallas import tpu as pltpu

_EPS = 1e-7

_CONTRACT_DIM1 = (((1,), (1,)), ((), ()))


def _stats_body(x_ref, m_ref, sc_ref, s_ref, q_ref, *, n_rows, n_steps):
    i = pl.program_id(1)

    @pl.when(i == 0)
    def _():
        s_ref[...] = jnp.zeros_like(s_ref)
        q_ref[...] = jnp.zeros_like(q_ref)

    xb = x_ref[...]
    s_ref[...] += jnp.sum(xb, axis=0, keepdims=True)
    q_ref[...] += jnp.sum(xb * xb, axis=0, keepdims=True)

    @pl.when(i == n_steps - 1)
    def _():
        s = s_ref[...]
        mean = s * (1.0 / n_rows)
        var = (q_ref[...] - s * mean) * (1.0 / (n_rows - 1.0))
        m_ref[...] = mean
        sc_ref[...] = 1.0 / (jnp.sqrt(jnp.maximum(var, 0.0)) + _EPS)


def _col_stats(x, rb, cb):
    n, d = x.shape
    grid = (d // cb, n // rb)
    return pl.pallas_call(
        functools.partial(_stats_body, n_rows=float(n), n_steps=n // rb),
        grid=grid,
        in_specs=[pl.BlockSpec((rb, cb), lambda c, i: (i, c))],
        out_specs=[
            pl.BlockSpec((1, cb), lambda c, i: (0, c)),
            pl.BlockSpec((1, cb), lambda c, i: (0, c)),
        ],
        out_shape=[
            jax.ShapeDtypeStruct((1, d), jnp.float32),
            jax.ShapeDtypeStruct((1, d), jnp.float32),
        ],
        scratch_shapes=[
            pltpu.VMEM((1, cb), jnp.float32),
            pltpu.VMEM((1, cb), jnp.float32),
        ],
        compiler_params=pltpu.CompilerParams(
            dimension_semantics=("parallel", "arbitrary")),
        name="col_stats",
    )(x)


def _dot_t(x, w):
    return jax.lax.dot_general(x, w, _CONTRACT_DIM1,
                               preferred_element_type=jnp.float32)


def _fused_body(u_ref, v_ref, mu_ref, scu_ref, mv_ref, scv_ref,
                wu0_ref, wv0_ref, b0_ref, wu1_ref, wv1_ref, b1_ref,
                ou_ref, ov_ref, *, xd):
    xu = (u_ref[...] - mu_ref[...]) * scu_ref[...]
    xv = (v_ref[...] - mv_ref[...]) * scv_ref[...]

    def layer(xu, xv, wu_ref, wv_ref, b_ref):
        h = _dot_t(xu, wv_ref[:, :xd]) + _dot_t(xv, wv_ref[:, xd:])
        pu = _dot_t(h, wu_ref[:xd, :]) + b_ref[:, :xd]
        pv = _dot_t(h, wu_ref[xd:, :]) + b_ref[:, xd:]
        return jnp.maximum(pu, 0.0) + xu, jnp.maximum(pv, 0.0) + xv

    xu, xv = layer(xu, xv, wu0_ref, wv0_ref, b0_ref)
    xu, xv = layer(xu, xv, wu1_ref, wv1_ref, b1_ref)
    ou_ref[...] = xu
    ov_ref[...] = xv


def kernel(u, v, wu0, wv0, b0, wu1, wv1, b1):
    u = u.astype(jnp.float32)
    v = v.astype(jnp.float32)
    n, xd = u.shape
    dd = v.shape[1]
    d = xd + dd

    rb_stats = min(2048, n)
    mu, scu = _col_stats(u, rb_stats, min(1024, xd))
    mv, scv = _col_stats(v, rb_stats, min(1024, dd))

    b0r = b0.reshape(1, d)
    b1r = b1.reshape(1, d)

    rb = min(256, n)
    grid = (n // rb,)
    full = lambda a: pl.BlockSpec(a.shape, lambda i: (0,) * a.ndim)
    ou, ov = pl.pallas_call(
        functools.partial(_fused_body, xd=xd),
        grid=grid,
        in_specs=[
            pl.BlockSpec((rb, xd), lambda i: (i, 0)),
            pl.BlockSpec((rb, dd), lambda i: (i, 0)),
            full(mu), full(scu), full(mv), full(scv),
            full(wu0), full(wv0), full(b0r),
            full(wu1), full(wv1), full(b1r),
        ],
        out_specs=[
            pl.BlockSpec((rb, xd), lambda i: (i, 0)),
            pl.BlockSpec((rb, dd), lambda i: (i, 0)),
        ],
        out_shape=[
            jax.ShapeDtypeStruct((n, xd), jnp.float32),
            jax.ShapeDtypeStruct((n, dd), jnp.float32),
        ],
        compiler_params=pltpu.CompilerParams(
            dimension_semantics=("parallel",),
            vmem_limit_bytes=56 * 1024 * 1024),
        name="norm_lr_mlp",
    )(u, v, mu, scu, mv, scv, wu0, wv0, b0r, wu1, wv1, b1r)
    return ou, ov
